# Initial kernel scaffold; baseline (speedup 1.0000x reference)
import jax
import jax.numpy as jnp
from jax.experimental import pallas as pl


def kernel(input_ids, embedding_table, position_table):
    raise NotImplementedError("write your pallas kernel here")


# speedup vs baseline: 1.3886x; 1.3886x over previous
import functools

import jax
import jax.numpy as jnp
from jax import lax
from jax.experimental import pallas as pl
from jax.experimental.pallas import tpu as pltpu
from jax.experimental.pallas import tpu_sc as plsc

NUM_CORES = 2
NUM_SUBCORES = 16
NUM_WORKERS = NUM_CORES * NUM_SUBCORES
LANES = 16
GATHER_SLICE = 100


def kernel(input_ids, embedding_table, position_table):
    B, T = input_ids.shape
    V, D = embedding_table.shape
    N = B * T

    ids = input_ids.reshape(N // GATHER_SLICE, GATHER_SLICE).astype(jnp.int32)
    pos = position_table[:T]

    rows_per_w = N // NUM_WORKERS
    chunk_seqs = 4
    C = chunk_seqs * T
    n_chunks = rows_per_w // C
    n_slices = C // GATHER_SLICE

    mesh = plsc.VectorSubcoreMesh(core_axis_name="c", subcore_axis_name="s")

    @functools.partial(
        pl.kernel,
        out_type=jax.ShapeDtypeStruct((N, D), jnp.float32),
        mesh=mesh,
        compiler_params=pltpu.CompilerParams(use_tc_tiling_on_sc=False),
        scratch_types=[
            pltpu.VMEM((n_slices, GATHER_SLICE), jnp.int32),
            pltpu.VMEM((C, D), jnp.float32),
            pltpu.VMEM((T, D), jnp.float32),
            pltpu.SemaphoreType.DMA,
        ],
    )
    def sc_embed(tab_hbm, ids_hbm, pos_hbm, out_hbm, idx_v, rows_v, pos_v, gsem):
        wid = lax.axis_index("s") * NUM_CORES + lax.axis_index("c")
        wbase = wid * rows_per_w
        pltpu.sync_copy(pos_hbm, pos_v)

        @pl.loop(0, n_chunks)
        def _(k):
            base = pl.multiple_of(wbase + k * C, 8 * GATHER_SLICE)
            idx_base = pl.multiple_of(base // GATHER_SLICE, 8)
            pltpu.sync_copy(ids_hbm.at[pl.ds(idx_base, n_slices)], idx_v)
            copies = [
                pltpu.async_copy(
                    tab_hbm.at[idx_v.at[j]],
                    rows_v.at[pl.ds(j * GATHER_SLICE, GATHER_SLICE)],
                    gsem,
                )
                for j in range(n_slices)
            ]
            for cp in copies:
                cp.wait()

            @pl.loop(0, T)
            def _(r):
                for c in range(0, D, LANES):
                    pvec = pos_v.at[pl.ds(r, 1), pl.ds(c, LANES)][...]
                    for s in range(chunk_seqs):
                        slc = (pl.ds(s * T + r, 1), pl.ds(c, LANES))
                        rows_v.at[slc][...] = rows_v.at[slc][...] + pvec

            pltpu.sync_copy(rows_v, out_hbm.at[pl.ds(base, C)])

    out = sc_embed(embedding_table, ids, pos)
    return out.reshape(B, T, D)


# speedup vs baseline: 1.4905x; 1.0734x over previous
import functools

import jax
import jax.numpy as jnp
from jax import lax
from jax.experimental import pallas as pl
from jax.experimental.pallas import tpu as pltpu
from jax.experimental.pallas import tpu_sc as plsc

NUM_CORES = 2
NUM_SUBCORES = 16
NUM_WORKERS = NUM_CORES * NUM_SUBCORES
LANES = 16
GATHER_SLICE = 100


def kernel(input_ids, embedding_table, position_table):
    B, T = input_ids.shape
    V, D = embedding_table.shape
    N = B * T

    ids = input_ids.reshape(N // GATHER_SLICE, GATHER_SLICE).astype(jnp.int32)
    pos = position_table[:T]

    rows_per_w = N // NUM_WORKERS
    chunk_seqs = 4
    C = chunk_seqs * T
    n_chunks = rows_per_w // C
    n_slices = C // GATHER_SLICE

    mesh = plsc.VectorSubcoreMesh(core_axis_name="c", subcore_axis_name="s")

    @functools.partial(
        pl.kernel,
        out_type=jax.ShapeDtypeStruct((N, D), jnp.float32),
        mesh=mesh,
        compiler_params=pltpu.CompilerParams(use_tc_tiling_on_sc=False),
        scratch_types=[
            pltpu.VMEM((n_slices, GATHER_SLICE), jnp.int32),
            pltpu.VMEM((n_slices, GATHER_SLICE), jnp.int32),
            pltpu.VMEM((C, D), jnp.float32),
            pltpu.VMEM((C, D), jnp.float32),
            pltpu.VMEM((T, D), jnp.float32),
            pltpu.SemaphoreType.DMA,
            pltpu.SemaphoreType.DMA,
            pltpu.SemaphoreType.DMA,
            pltpu.SemaphoreType.DMA,
            pltpu.SemaphoreType.DMA,
            pltpu.SemaphoreType.DMA,
        ],
    )
    def sc_embed(tab_hbm, ids_hbm, pos_hbm, out_hbm,
                 idx0, idx1, rows0, rows1, pos_v,
                 g0, g1, i0, i1, o0, o1):
        idxb, rowsb = [idx0, idx1], [rows0, rows1]
        gsem, isem, osem = [g0, g1], [i0, i1], [o0, o1]
        wid = lax.axis_index("s") * NUM_CORES + lax.axis_index("c")
        wbase = wid * rows_per_w
        pltpu.sync_copy(pos_hbm, pos_v)

        def idx_base(k):
            return pl.multiple_of((wbase + k * C) // GATHER_SLICE, 8)

        def row_base(k):
            return pl.multiple_of(wbase + k * C, 8)

        def fire_idx(k, b):
            pltpu.async_copy(ids_hbm.at[pl.ds(idx_base(k), n_slices)], idxb[b], isem[b])

        def wait_idx(b):
            pltpu.make_async_copy(ids_hbm.at[pl.ds(0, n_slices)], idxb[b], isem[b]).wait()

        def fire_gathers(b):
            for j in range(n_slices):
                pltpu.async_copy(
                    tab_hbm.at[idxb[b].at[j]],
                    rowsb[b].at[pl.ds(j * GATHER_SLICE, GATHER_SLICE)],
                    gsem[b],
                )

        def wait_gathers(b):
            pltpu.make_async_copy(out_hbm.at[pl.ds(0, C)], rowsb[b], gsem[b]).wait()

        def fire_out(k, b):
            pltpu.async_copy(rowsb[b], out_hbm.at[pl.ds(row_base(k), C)], osem[b])

        def wait_out(b):
            pltpu.make_async_copy(rowsb[b], out_hbm.at[pl.ds(0, C)], osem[b]).wait()

        def add_pos(b):
            @pl.loop(0, T)
            def _(r):
                for c in range(0, D, LANES):
                    pvec = pos_v.at[pl.ds(r, 1), pl.ds(c, LANES)][...]
                    for s in range(chunk_seqs):
                        slc = (pl.ds(s * T + r, 1), pl.ds(c, LANES))
                        rowsb[b].at[slc][...] = rowsb[b].at[slc][...] + pvec

        pltpu.sync_copy(ids_hbm.at[pl.ds(idx_base(0), n_slices)], idxb[0])
        fire_gathers(0)
        fire_idx(1, 1)

        @pl.loop(0, n_chunks, step=2)
        def _(k0):
            for b in range(2):
                k = k0 + b
                nb = 1 - b

                @pl.when(k + 1 < n_chunks)
                def _():
                    wait_idx(nb)

                    @pl.when(k >= 1)
                    def _():
                        wait_out(nb)

                    fire_gathers(nb)

                wait_gathers(b)

                @pl.when(k + 2 < n_chunks)
                def _():
                    fire_idx(k + 2, b)

                add_pos(b)
                fire_out(k, b)

        wait_out(0)
        wait_out(1)

    out = sc_embed(embedding_table, ids, pos)
    return out.reshape(B, T, D)
